# baseline (device time: 35039 ns/iter reference)
import jax
import jax.numpy as jnp
from jax import lax
from jax.experimental import pallas as pl
from jax.experimental.pallas import tpu as pltpu

N_DEV = 4


def kernel(x, k, Wp):
    b, s, c = x.shape
    n = Wp.shape[1]
    bs = b * s

    def body(x_ref, k_ref, wp_ref, out_ref, sa, ra, sb, rb, send_sems, recv_sems):
        my = lax.axis_index("i")
        pa = my ^ 1
        pb = 3 - my

        barrier_sem = pltpu.get_barrier_semaphore()
        for nbr in (pa, pb):
            pl.semaphore_signal(
                barrier_sem, inc=1,
                device_id=(nbr,), device_id_type=pl.DeviceIdType.MESH,
            )
        pl.semaphore_wait(barrier_sem, 2)

        xv = x_ref[...]
        kv = k_ref[...]
        acc = xv * kv[3][None, None, :]
        for shift in range(1, 4):
            shifted = jnp.concatenate(
                [jnp.zeros((b, shift, c), xv.dtype), xv[:, : s - shift, :]],
                axis=1,
            )
            acc = acc + shifted * kv[3 - shift][None, None, :]
        a = (acc * jax.nn.sigmoid(acc)).astype(jnp.bfloat16)
        wv = wp_ref[...].astype(jnp.bfloat16)
        partial = lax.dot_general(
            a.reshape(bs, c), wv,
            dimension_numbers=(((1,), (0,)), ((), ())),
            preferred_element_type=jnp.float32,
        )
        sa[...] = partial.astype(jnp.bfloat16)

        ex_a = pltpu.make_async_remote_copy(
            src_ref=sa, dst_ref=ra,
            send_sem=send_sems.at[0], recv_sem=recv_sems.at[0],
            device_id=(pa,), device_id_type=pl.DeviceIdType.MESH,
        )
        ex_a.start()
        ex_a.wait()

        sb[...] = sa[...] + ra[...]

        ex_b = pltpu.make_async_remote_copy(
            src_ref=sb, dst_ref=rb,
            send_sem=send_sems.at[1], recv_sem=recv_sems.at[1],
            device_id=(pb,), device_id_type=pl.DeviceIdType.MESH,
        )
        ex_b.start()
        ex_b.wait()

        total = sb[...].astype(jnp.float32) + rb[...].astype(jnp.float32)
        out_ref[...] = total.reshape(b, s, n)

    return pl.pallas_call(
        body,
        out_shape=jax.ShapeDtypeStruct((b, s, n), jnp.float32),
        in_specs=[pl.BlockSpec(memory_space=pltpu.VMEM)] * 3,
        out_specs=pl.BlockSpec(memory_space=pltpu.VMEM),
        scratch_shapes=[
            pltpu.VMEM((bs, n), jnp.bfloat16),
            pltpu.VMEM((bs, n), jnp.bfloat16),
            pltpu.VMEM((bs, n), jnp.bfloat16),
            pltpu.VMEM((bs, n), jnp.bfloat16),
            pltpu.SemaphoreType.DMA((2,)),
            pltpu.SemaphoreType.DMA((2,)),
        ],
        compiler_params=pltpu.CompilerParams(collective_id=0),
    )(x, k, Wp)


# device time: 23835 ns/iter; 1.4701x vs baseline; 1.4701x over previous
import jax
import jax.numpy as jnp
from jax import lax
from jax.experimental import pallas as pl
from jax.experimental.pallas import tpu as pltpu

N_DEV = 4


def kernel(x, k, Wp):
    b, s, c = x.shape
    n = Wp.shape[1]
    bs = b * s
    h = bs // 2

    def body(x_ref, k_ref, wp_ref, out_ref,
             sa1, ra1, sb1, rb1, sa2, ra2, sb2, rb2,
             send_sems, recv_sems):
        my = lax.axis_index("i")
        pa = my ^ 1
        pb = 3 - my

        barrier_sem = pltpu.get_barrier_semaphore()
        for nbr in (pa, pb):
            pl.semaphore_signal(
                barrier_sem, inc=1,
                device_id=(nbr,), device_id_type=pl.DeviceIdType.MESH,
            )
        pl.semaphore_wait(barrier_sem, 2)

        xv = x_ref[...]
        kv = k_ref[...]
        acc = xv * kv[3][None, None, :]
        for shift in range(1, 4):
            shifted = jnp.concatenate(
                [jnp.zeros((b, shift, c), xv.dtype), xv[:, : s - shift, :]],
                axis=1,
            )
            acc = acc + shifted * kv[3 - shift][None, None, :]
        a = (acc * jax.nn.sigmoid(acc)).astype(jnp.bfloat16)
        wv = wp_ref[...].astype(jnp.bfloat16)
        partial = lax.dot_general(
            a.reshape(bs, c), wv,
            dimension_numbers=(((1,), (0,)), ((), ())),
            preferred_element_type=jnp.float32,
        ).astype(jnp.bfloat16)

        sa1[...] = partial[:h]
        sb2[...] = partial[h:]

        ex_a1 = pltpu.make_async_remote_copy(
            src_ref=sa1, dst_ref=ra1,
            send_sem=send_sems.at[0], recv_sem=recv_sems.at[0],
            device_id=(pa,), device_id_type=pl.DeviceIdType.MESH,
        )
        ex_b2 = pltpu.make_async_remote_copy(
            src_ref=sb2, dst_ref=rb2,
            send_sem=send_sems.at[1], recv_sem=recv_sems.at[1],
            device_id=(pb,), device_id_type=pl.DeviceIdType.MESH,
        )
        ex_a1.start()
        ex_b2.start()
        ex_a1.wait()
        ex_b2.wait()

        sb1[...] = sa1[...] + ra1[...]
        sa2[...] = sb2[...] + rb2[...]

        ex_b1 = pltpu.make_async_remote_copy(
            src_ref=sb1, dst_ref=rb1,
            send_sem=send_sems.at[2], recv_sem=recv_sems.at[2],
            device_id=(pb,), device_id_type=pl.DeviceIdType.MESH,
        )
        ex_a2 = pltpu.make_async_remote_copy(
            src_ref=sa2, dst_ref=ra2,
            send_sem=send_sems.at[3], recv_sem=recv_sems.at[3],
            device_id=(pa,), device_id_type=pl.DeviceIdType.MESH,
        )
        ex_b1.start()
        ex_a2.start()
        ex_b1.wait()
        ex_a2.wait()

        t1 = sb1[...].astype(jnp.float32) + rb1[...].astype(jnp.float32)
        t2 = sa2[...].astype(jnp.float32) + ra2[...].astype(jnp.float32)
        out_ref[...] = jnp.concatenate([t1, t2], axis=0).reshape(b, s, n)

    half = (h, n)
    return pl.pallas_call(
        body,
        out_shape=jax.ShapeDtypeStruct((b, s, n), jnp.float32),
        in_specs=[pl.BlockSpec(memory_space=pltpu.VMEM)] * 3,
        out_specs=pl.BlockSpec(memory_space=pltpu.VMEM),
        scratch_shapes=[
            pltpu.VMEM(half, jnp.bfloat16),
            pltpu.VMEM(half, jnp.bfloat16),
            pltpu.VMEM(half, jnp.bfloat16),
            pltpu.VMEM(half, jnp.bfloat16),
            pltpu.VMEM(half, jnp.bfloat16),
            pltpu.VMEM(half, jnp.bfloat16),
            pltpu.VMEM(half, jnp.bfloat16),
            pltpu.VMEM(half, jnp.bfloat16),
            pltpu.SemaphoreType.DMA((4,)),
            pltpu.SemaphoreType.DMA((4,)),
        ],
        compiler_params=pltpu.CompilerParams(collective_id=0),
    )(x, k, Wp)


# device time: 23058 ns/iter; 1.5196x vs baseline; 1.0337x over previous
import jax
import jax.numpy as jnp
from jax import lax
from jax.experimental import pallas as pl
from jax.experimental.pallas import tpu as pltpu

N_DEV = 4


def kernel(x, k, Wp):
    b, s, c = x.shape
    n = Wp.shape[1]
    bs = b * s
    h = bs // 2

    def body(x_ref, k_ref, wp_ref, out_ref,
             sa1, ra1, sb1, rb1, sa2, ra2, sb2, rb2,
             send_sems, recv_sems):
        my = lax.axis_index("i")
        pa = my ^ 1
        pb = 3 - my

        barrier_sem = pltpu.get_barrier_semaphore()
        for nbr in (pa, pb):
            pl.semaphore_signal(
                barrier_sem, inc=1,
                device_id=(nbr,), device_id_type=pl.DeviceIdType.MESH,
            )

        kv = k_ref[...].astype(jnp.bfloat16)
        wv = wp_ref[...].astype(jnp.bfloat16)

        def half_partial(lo):
            xv = x_ref[lo:lo + 2].astype(jnp.bfloat16)
            acc = xv * kv[3][None, None, :]
            for shift in range(1, 4):
                shifted = jnp.concatenate(
                    [jnp.zeros((2, shift, c), xv.dtype), xv[:, : s - shift, :]],
                    axis=1,
                )
                acc = acc + shifted * kv[3 - shift][None, None, :]
            a = acc * jax.nn.sigmoid(acc)
            return lax.dot_general(
                a.reshape(h, c), wv,
                dimension_numbers=(((1,), (0,)), ((), ())),
                preferred_element_type=jnp.float32,
            ).astype(jnp.bfloat16)

        ex_a1 = pltpu.make_async_remote_copy(
            src_ref=sa1, dst_ref=ra1,
            send_sem=send_sems.at[0], recv_sem=recv_sems.at[0],
            device_id=(pa,), device_id_type=pl.DeviceIdType.MESH,
        )
        ex_b2 = pltpu.make_async_remote_copy(
            src_ref=sb2, dst_ref=rb2,
            send_sem=send_sems.at[1], recv_sem=recv_sems.at[1],
            device_id=(pb,), device_id_type=pl.DeviceIdType.MESH,
        )
        ex_b1 = pltpu.make_async_remote_copy(
            src_ref=sb1, dst_ref=rb1,
            send_sem=send_sems.at[2], recv_sem=recv_sems.at[2],
            device_id=(pb,), device_id_type=pl.DeviceIdType.MESH,
        )
        ex_a2 = pltpu.make_async_remote_copy(
            src_ref=sa2, dst_ref=ra2,
            send_sem=send_sems.at[3], recv_sem=recv_sems.at[3],
            device_id=(pa,), device_id_type=pl.DeviceIdType.MESH,
        )

        sa1[...] = half_partial(0)
        pl.semaphore_wait(barrier_sem, 2)
        ex_a1.start()
        sb2[...] = half_partial(2)
        ex_b2.start()

        ex_a1.wait()
        sb1[...] = sa1[...] + ra1[...]
        ex_b1.start()
        ex_b2.wait()
        sa2[...] = sb2[...] + rb2[...]
        ex_a2.start()

        ex_b1.wait()
        t1 = sb1[...].astype(jnp.float32) + rb1[...].astype(jnp.float32)
        out_ref[0:2] = t1.reshape(2, s, n)
        ex_a2.wait()
        t2 = sa2[...].astype(jnp.float32) + ra2[...].astype(jnp.float32)
        out_ref[2:4] = t2.reshape(2, s, n)

    half = (h, n)
    return pl.pallas_call(
        body,
        out_shape=jax.ShapeDtypeStruct((b, s, n), jnp.float32),
        in_specs=[pl.BlockSpec(memory_space=pltpu.VMEM)] * 3,
        out_specs=pl.BlockSpec(memory_space=pltpu.VMEM),
        scratch_shapes=[
            pltpu.VMEM(half, jnp.bfloat16),
            pltpu.VMEM(half, jnp.bfloat16),
            pltpu.VMEM(half, jnp.bfloat16),
            pltpu.VMEM(half, jnp.bfloat16),
            pltpu.VMEM(half, jnp.bfloat16),
            pltpu.VMEM(half, jnp.bfloat16),
            pltpu.VMEM(half, jnp.bfloat16),
            pltpu.VMEM(half, jnp.bfloat16),
            pltpu.SemaphoreType.DMA((4,)),
            pltpu.SemaphoreType.DMA((4,)),
        ],
        compiler_params=pltpu.CompilerParams(collective_id=0),
    )(x, k, Wp)


# device time: 20525 ns/iter; 1.7071x vs baseline; 1.1234x over previous
import jax
import jax.numpy as jnp
from jax import lax
from jax.experimental import pallas as pl
from jax.experimental.pallas import tpu as pltpu

N_DEV = 4


def kernel(x, k, Wp):
    b, s, c = x.shape
    n = Wp.shape[1]
    bs = b * s
    h = bs // 2

    def body(x_ref, k_ref, wp_ref, out_ref,
             p1s, p1r, p2s, p2r,
             send_sems, recv_sems):
        my = lax.axis_index("i")
        pa = my ^ 1
        pb = 3 - my

        barrier_sem = pltpu.get_barrier_semaphore()
        for nbr in (pa, pb):
            pl.semaphore_signal(
                barrier_sem, inc=1,
                device_id=(nbr,), device_id_type=pl.DeviceIdType.MESH,
            )

        kv = k_ref[...].astype(jnp.bfloat16)
        wv = wp_ref[...].astype(jnp.bfloat16)

        def chunk_partial(i):
            xv = x_ref[i].astype(jnp.bfloat16)
            acc = xv * kv[3][None, :]
            for shift in range(1, 4):
                shifted = jnp.concatenate(
                    [jnp.zeros((shift, c), xv.dtype), xv[: s - shift, :]],
                    axis=0,
                )
                acc = acc + shifted * kv[3 - shift][None, :]
            a = acc * jax.nn.sigmoid(acc)
            return lax.dot_general(
                a, wv,
                dimension_numbers=(((1,), (0,)), ((), ())),
                preferred_element_type=jnp.float32,
            ).astype(jnp.bfloat16)

        p1_dst = {0: pa, 1: pa, 2: pb, 3: pb}
        p2_dst = {0: pb, 1: pb, 2: pa, 3: pa}
        ex1 = {
            i: pltpu.make_async_remote_copy(
                src_ref=p1s.at[i], dst_ref=p1r.at[i],
                send_sem=send_sems.at[i], recv_sem=recv_sems.at[i],
                device_id=(p1_dst[i],), device_id_type=pl.DeviceIdType.MESH,
            )
            for i in range(4)
        }
        ex2 = {
            i: pltpu.make_async_remote_copy(
                src_ref=p2s.at[i], dst_ref=p2r.at[i],
                send_sem=send_sems.at[4 + i], recv_sem=recv_sems.at[4 + i],
                device_id=(p2_dst[i],), device_id_type=pl.DeviceIdType.MESH,
            )
            for i in range(4)
        }

        order = (0, 2, 1, 3)
        p1s[order[0]] = chunk_partial(order[0])
        pl.semaphore_wait(barrier_sem, 2)
        ex1[order[0]].start()
        for i in order[1:]:
            p1s[i] = chunk_partial(i)
            ex1[i].start()

        for i in order:
            ex1[i].wait()
            p2s[i] = p1s[i] + p1r[i]
            ex2[i].start()

        for i in order:
            ex2[i].wait()
            out_ref[i] = (
                p2s[i].astype(jnp.float32) + p2r[i].astype(jnp.float32)
            ).reshape(s, n)

    half = (h, n)
    return pl.pallas_call(
        body,
        out_shape=jax.ShapeDtypeStruct((b, s, n), jnp.float32),
        in_specs=[pl.BlockSpec(memory_space=pltpu.VMEM)] * 3,
        out_specs=pl.BlockSpec(memory_space=pltpu.VMEM),
        scratch_shapes=[
            pltpu.VMEM((4, s, n), jnp.bfloat16),
            pltpu.VMEM((4, s, n), jnp.bfloat16),
            pltpu.VMEM((4, s, n), jnp.bfloat16),
            pltpu.VMEM((4, s, n), jnp.bfloat16),
            pltpu.SemaphoreType.DMA((8,)),
            pltpu.SemaphoreType.DMA((8,)),
        ],
        compiler_params=pltpu.CompilerParams(collective_id=0),
    )(x, k, Wp)
